# Initial kernel scaffold; baseline (speedup 1.0000x reference)
import jax
import jax.numpy as jnp
from jax.experimental import pallas as pl


def kernel(x, edge_index, W1, b1, W2, b2, Wh1, bh1, Wh2, bh2):
    raise NotImplementedError("write your pallas kernel here")


# speedup vs baseline: 11.6275x; 11.6275x over previous
import functools

import jax
import jax.numpy as jnp
from jax import lax
from jax.experimental import pallas as pl
from jax.experimental.pallas import tpu as pltpu
from jax.experimental.pallas import tpu_sc as plsc

N = 10000
E = 320000
IN_CH = 128
HID = 64
W128 = 128

NC = 2
NS = 16
NW = NC * NS
CHUNK = 128
CPW = -(-E // (NW * CHUNK))
EPW = CPW * CHUNK
EPAD = NW * EPW
RPT = 640
ACC_ROWS = NS * RPT
MB = 1000

_MESH = plsc.VectorSubcoreMesh(
    core_axis_name="c", subcore_axis_name="s", num_cores=NC, num_subcores=NS
)


@functools.partial(
    pl.kernel,
    out_type=jax.ShapeDtypeStruct((NC * ACC_ROWS, W128), jnp.float32),
    mesh=_MESH,
    scratch_types=[
        pltpu.VMEM((CHUNK,), jnp.int32),
        pltpu.VMEM((CHUNK, W128), jnp.float32),
        pltpu.VMEM_SHARED((ACC_ROWS, W128), jnp.float32),
    ],
)
def _sc_degree(dst_hbm, ones_hbm, zeros_hbm, out_hbm, idx_v, ones_v, acc):
    c = lax.axis_index("c")
    s = lax.axis_index("s")
    wid = s * NC + c
    pltpu.sync_copy(zeros_hbm, acc.at[pl.ds(s * RPT, RPT)])
    pltpu.sync_copy(ones_hbm, ones_v)
    plsc.subcore_barrier()
    base = wid * EPW

    def body(j, carry):
        start = base + j * CHUNK
        pltpu.sync_copy(dst_hbm.at[pl.ds(start, CHUNK)], idx_v)
        pltpu.sync_copy(ones_v, acc.at[idx_v], add=True)
        return carry

    lax.fori_loop(0, CPW, body, 0)
    plsc.subcore_barrier()
    pltpu.sync_copy(
        acc.at[pl.ds(s * RPT, RPT)],
        out_hbm.at[pl.ds(c * ACC_ROWS + s * RPT, RPT)],
    )


@functools.partial(
    pl.kernel,
    out_type=jax.ShapeDtypeStruct((NC * ACC_ROWS, W128), jnp.float32),
    mesh=_MESH,
    scratch_types=[
        pltpu.VMEM((CHUNK,), jnp.int32),
        pltpu.VMEM((CHUNK,), jnp.int32),
        pltpu.VMEM((CHUNK, W128), jnp.float32),
        pltpu.VMEM_SHARED((ACC_ROWS, W128), jnp.float32),
        pltpu.SemaphoreType.DMA,
    ],
)
def _sc_agg(h_hbm, src_hbm, dst_hbm, zeros_hbm, out_hbm, sidx, didx, rows, acc, sem):
    c = lax.axis_index("c")
    s = lax.axis_index("s")
    wid = s * NC + c
    pltpu.sync_copy(zeros_hbm, acc.at[pl.ds(s * RPT, RPT)])
    plsc.subcore_barrier()
    base = wid * EPW

    def body(j, carry):
        start = base + j * CHUNK
        pltpu.sync_copy(src_hbm.at[pl.ds(start, CHUNK)], sidx)
        gather = pltpu.async_copy(h_hbm.at[sidx], rows, sem)
        pltpu.sync_copy(dst_hbm.at[pl.ds(start, CHUNK)], didx)
        gather.wait()
        pltpu.sync_copy(rows, acc.at[didx], add=True)
        return carry

    lax.fori_loop(0, CPW, body, 0)
    plsc.subcore_barrier()
    pltpu.sync_copy(
        acc.at[pl.ds(s * RPT, RPT)],
        out_hbm.at[pl.ds(c * ACC_ROWS + s * RPT, RPT)],
    )


def _dinv_of(degp_ref):
    deg = degp_ref[0, :, :1] + degp_ref[1, :, :1] + 1.0
    return lax.rsqrt(deg)


def _mm_prescale_body(x_ref, w_ref, degp_ref, out_ref):
    h = jnp.dot(x_ref[...], w_ref[...], preferred_element_type=jnp.float32)
    out_ref[...] = h * _dinv_of(degp_ref)


def _mid_body(aggp_ref, hp_ref, degp_ref, b_ref, w_ref, out_ref):
    dinv = _dinv_of(degp_ref)
    u = (aggp_ref[0] + aggp_ref[1] + hp_ref[...]) * dinv + b_ref[...]
    t = jnp.maximum(u, 0.0)
    h = jnp.dot(t, w_ref[...], preferred_element_type=jnp.float32)
    out_ref[...] = h * dinv


def _tail_body(aggp_ref, hp_ref, degp_ref, b2_ref, wh1_ref, bh1_ref,
               wh2_ref, bh2_ref, out_ref):
    dinv = _dinv_of(degp_ref)
    u = (aggp_ref[0] + aggp_ref[1] + hp_ref[...]) * dinv + b2_ref[...]
    g = jnp.maximum(u, 0.0)
    t = jnp.dot(g, wh1_ref[...], preferred_element_type=jnp.float32)
    t = jnp.maximum(t + bh1_ref[...], 0.0)
    out_ref[...] = jnp.dot(t, wh2_ref[...], preferred_element_type=jnp.float32) + bh2_ref[...]


def _row_spec(width):
    return pl.BlockSpec((MB, width), lambda i: (i, 0))


def _full_spec(shape):
    nd = len(shape)
    return pl.BlockSpec(shape, lambda i: (0,) * nd)


_PART_SPEC = pl.BlockSpec((2, MB, W128), lambda i: (0, i, 0))

_GRID = (N // MB,)

_mm_prescale = pl.pallas_call(
    _mm_prescale_body,
    grid=_GRID,
    in_specs=[_row_spec(IN_CH), _full_spec((IN_CH, W128)), _PART_SPEC],
    out_specs=_row_spec(W128),
    out_shape=jax.ShapeDtypeStruct((N, W128), jnp.float32),
)

_mid = pl.pallas_call(
    _mid_body,
    grid=_GRID,
    in_specs=[_PART_SPEC, _row_spec(W128), _PART_SPEC,
              _full_spec((1, W128)), _full_spec((W128, W128))],
    out_specs=_row_spec(W128),
    out_shape=jax.ShapeDtypeStruct((N, W128), jnp.float32),
)

_tail = pl.pallas_call(
    _tail_body,
    grid=_GRID,
    in_specs=[_PART_SPEC, _row_spec(W128), _PART_SPEC,
              _full_spec((1, W128)), _full_spec((W128, W128)),
              _full_spec((1, W128)), _full_spec((W128, W128)),
              _full_spec((1, W128))],
    out_specs=_row_spec(W128),
    out_shape=jax.ShapeDtypeStruct((N, W128), jnp.float32),
)


def _padw(a, rows=None):
    r = a.shape[0] if rows is None else rows
    out = jnp.zeros((r, W128), jnp.float32)
    return out.at[: a.shape[0], : a.shape[1]].set(a)


def kernel(x, edge_index, W1, b1, W2, b2, Wh1, bh1, Wh2, bh2):
    ei = edge_index.astype(jnp.int32)
    pad = EPAD - E
    src_p = jnp.concatenate([ei[0], jnp.zeros((pad,), jnp.int32)])
    dst_p = jnp.concatenate([ei[1], jnp.full((pad,), N, jnp.int32)])

    ones128 = jnp.ones((CHUNK, W128), jnp.float32)
    zeros128 = jnp.zeros((RPT, W128), jnp.float32)

    degp = _sc_degree(dst_p, ones128, zeros128).reshape(NC, ACC_ROWS, W128)

    h1p = _mm_prescale(x, _padw(W1), degp)
    agg1 = _sc_agg(h1p, src_p, dst_p, zeros128).reshape(NC, ACC_ROWS, W128)

    h2p = _mid(agg1, h1p, degp, _padw(b1.reshape(1, HID)), _padw(W2, W128))
    agg2 = _sc_agg(h2p, src_p, dst_p, zeros128).reshape(NC, ACC_ROWS, W128)

    y = _tail(agg2, h2p, degp, _padw(b2.reshape(1, HID)),
              _padw(Wh1, W128), _padw(bh1.reshape(1, HID // 2)),
              _padw(Wh2, W128), _padw(bh2.reshape(1, 2)))
    return y[:, :2]
